# Initial kernel scaffold; baseline (speedup 1.0000x reference)
import functools

import jax
import jax.numpy as jnp
from jax import lax
from jax.experimental import pallas as pl
from jax.experimental.pallas import tpu as pltpu
from jax.experimental.pallas import tpu_sc as plsc

N_NODES = 10000
N_PAD = 10240
D = 128
N_GRAPHS = 64
NC, NS = 2, 16
NW = NC * NS
CHUNK = 128
CPT = 80
E_PAD = NW * CPT * CHUNK
ROWS_PER_TILE = N_PAD // NS
BLK = 512
GRID = N_PAD // BLK


def _make_sc_segsum(with_cnt):
  mesh = plsc.VectorSubcoreMesh(core_axis_name="c", subcore_axis_name="s")
  out_type = [jax.ShapeDtypeStruct((NC, N_PAD, D), jnp.float32)]
  scratch = [
      pltpu.VMEM((CPT, CHUNK), jnp.int32),
      pltpu.VMEM((CPT, CHUNK), jnp.int32),
      pltpu.VMEM((CHUNK, D), jnp.float32),
      pltpu.VMEM_SHARED((N_PAD, D), jnp.float32),
      pltpu.SemaphoreType.DMA,
  ]
  if with_cnt:
    out_type.append(jax.ShapeDtypeStruct((NC, N_PAD, 16), jnp.float32))
    scratch += [
        pltpu.VMEM((CHUNK, 16), jnp.float32),
        pltpu.VMEM_SHARED((N_PAD, 16), jnp.float32),
    ]

  def body(h_hbm, src_hbm, dst_hbm, zeros_hbm, *rest):
    if with_cnt:
      (ones_hbm, out_hbm, cnt_hbm, src_v, dst_v, rows_v, acc, sem,
       ones_v, cacc) = rest
    else:
      out_hbm, src_v, dst_v, rows_v, acc, sem = rest
    cid = lax.axis_index("c")
    sid = lax.axis_index("s")
    row0 = sid * ROWS_PER_TILE

    pltpu.sync_copy(src_hbm.at[cid, sid], src_v)
    pltpu.sync_copy(dst_hbm.at[cid, sid], dst_v)
    pltpu.sync_copy(zeros_hbm.at[pl.ds(row0, ROWS_PER_TILE)],
                    acc.at[pl.ds(row0, ROWS_PER_TILE)])
    if with_cnt:
      pltpu.sync_copy(zeros_hbm.at[pl.ds(row0, ROWS_PER_TILE), pl.ds(0, 16)],
                      cacc.at[pl.ds(row0, ROWS_PER_TILE)])
    plsc.subcore_barrier()

    def chunk(i, carry):
      pltpu.async_copy(h_hbm.at[src_v.at[i]], rows_v, sem).wait()
      pltpu.sync_copy(rows_v, acc.at[dst_v.at[i]], add=True)
      if with_cnt:
        pltpu.async_copy(ones_hbm.at[src_v.at[i]], ones_v, sem).wait()
        pltpu.sync_copy(ones_v, cacc.at[dst_v.at[i]], add=True)
      return carry

    lax.fori_loop(0, CPT, chunk, 0)
    plsc.subcore_barrier()

    pltpu.sync_copy(acc.at[pl.ds(row0, ROWS_PER_TILE)],
                    out_hbm.at[cid, pl.ds(row0, ROWS_PER_TILE)])
    if with_cnt:
      pltpu.sync_copy(cacc.at[pl.ds(row0, ROWS_PER_TILE)],
                      cnt_hbm.at[cid, pl.ds(row0, ROWS_PER_TILE)])

  return pl.kernel(body, out_type=tuple(out_type) if with_cnt else out_type[0],
                   mesh=mesh, scratch_types=scratch)


_sc_segsum_cnt = _make_sc_segsum(True)
_sc_segsum = _make_sc_segsum(False)


def _layer_body(p_ref, c_ref, x_ref, wl_ref, wr_ref, bl_ref, o_ref):
  i = pl.program_id(0)
  msg = p_ref[0] + p_ref[1]
  cnt = (c_ref[0] + c_ref[1])[:, 0:1]
  inv = 1.0 / jnp.maximum(cnt, 1.0)
  h = inv * jnp.dot(msg, wl_ref[...], preferred_element_type=jnp.float32)
  h = h + bl_ref[...] + jnp.dot(x_ref[...], wr_ref[...],
                                preferred_element_type=jnp.float32)
  h = jnp.maximum(h, 0.0)
  rowid = i * BLK + lax.broadcasted_iota(jnp.int32, (BLK, 1), 0)
  o_ref[...] = jnp.where(rowid < N_NODES, h, 0.0)


_tc_layer = pl.pallas_call(
    _layer_body,
    grid=(GRID,),
    in_specs=[
        pl.BlockSpec((NC, BLK, D), lambda i: (0, i, 0)),
        pl.BlockSpec((NC, BLK, 16), lambda i: (0, i, 0)),
        pl.BlockSpec((BLK, D), lambda i: (i, 0)),
        pl.BlockSpec((D, D), lambda i: (0, 0)),
        pl.BlockSpec((D, D), lambda i: (0, 0)),
        pl.BlockSpec((1, D), lambda i: (0, 0)),
    ],
    out_specs=pl.BlockSpec((BLK, D), lambda i: (i, 0)),
    out_shape=jax.ShapeDtypeStruct((N_PAD, D), jnp.float32),
)


def _final_body(h_ref, b_ref, g_ref, beta_ref, wd_ref, bd_ref, o_ref, pool):
  i = pl.program_id(0)

  @pl.when(i == 0)
  def _():
    pool[...] = jnp.zeros_like(pool)

  b = b_ref[...]
  onehot = (b == lax.broadcasted_iota(jnp.int32, (1, N_GRAPHS), 1)
            ).astype(jnp.float32)
  pool[...] += lax.dot_general(onehot, h_ref[...],
                               (((0,), (0,)), ((), ())),
                               preferred_element_type=jnp.float32)

  @pl.when(i == GRID - 1)
  def _():
    p = pool[...]
    m = jnp.mean(p, axis=1, keepdims=True)
    v = jnp.mean((p - m) * (p - m), axis=1, keepdims=True)
    ln = (p - m) * lax.rsqrt(v + 1e-5) * g_ref[...] + beta_ref[...]
    o_ref[...] = jnp.dot(ln, wd_ref[...],
                         preferred_element_type=jnp.float32) + bd_ref[...]


_tc_final = pl.pallas_call(
    _final_body,
    grid=(GRID,),
    in_specs=[
        pl.BlockSpec((BLK, D), lambda i: (i, 0)),
        pl.BlockSpec((BLK, 1), lambda i: (i, 0)),
        pl.BlockSpec((1, D), lambda i: (0, 0)),
        pl.BlockSpec((1, D), lambda i: (0, 0)),
        pl.BlockSpec((D, D), lambda i: (0, 0)),
        pl.BlockSpec((1, D), lambda i: (0, 0)),
    ],
    out_specs=pl.BlockSpec((N_GRAPHS, D), lambda i: (0, 0)),
    out_shape=jax.ShapeDtypeStruct((N_GRAPHS, D), jnp.float32),
    scratch_shapes=[pltpu.VMEM((N_GRAPHS, D), jnp.float32)],
)


def kernel(x, edge_index, batch, Wl0, Wr0, bl0, Wl1, Wr1, bl1,
           Wl2, Wr2, bl2, ln_g, ln_b, Wd, bd):
  f32 = jnp.float32
  x_pad = jnp.zeros((N_PAD, D), f32).at[:N_NODES].set(x)
  valid = (jnp.arange(N_PAD) < N_NODES).astype(f32)
  ones_tab = jnp.broadcast_to(valid[:, None], (N_PAD, 16))
  zeros_tab = jnp.zeros((N_PAD, D), f32)

  src = jnp.full((E_PAD,), N_NODES, jnp.int32).at[:edge_index.shape[1]].set(
      edge_index[0])
  dst = jnp.zeros((E_PAD,), jnp.int32).at[:edge_index.shape[1]].set(
      edge_index[1])
  src_r = src.reshape(NC, NS, CPT, CHUNK)
  dst_r = dst.reshape(NC, NS, CPT, CHUNK)

  batch2 = jnp.zeros((N_PAD, 1), jnp.int32).at[:N_NODES, 0].set(batch)

  p0, c0 = _sc_segsum_cnt(x_pad, src_r, dst_r, zeros_tab, ones_tab)
  h = _tc_layer(p0, c0, x_pad, Wl0.T, Wr0.T, bl0[None])
  p1 = _sc_segsum(h, src_r, dst_r, zeros_tab)
  h = _tc_layer(p1, c0, h, Wl1.T, Wr1.T, bl1[None])
  p2 = _sc_segsum(h, src_r, dst_r, zeros_tab)
  h = _tc_layer(p2, c0, h, Wl2.T, Wr2.T, bl2[None])
  return _tc_final(h, batch2, ln_g[None], ln_b[None], Wd.T, bd[None])


# speedup vs baseline: 2.6056x; 2.6056x over previous
import jax
import jax.numpy as jnp
from jax import lax
from jax.experimental import pallas as pl
from jax.experimental.pallas import tpu as pltpu
from jax.experimental.pallas import tpu_sc as plsc

N_NODES = 10000
N_PAD = 10240
D = 128
N_GRAPHS = 64
NC, NS = 2, 16
NW = NC * NS
CHUNK = 128
CPT = 80
E_PAD = NW * CPT * CHUNK
ROWS_PER_TILE = N_PAD // NS
BLK = 512
GRID = N_PAD // BLK

_MESH = plsc.VectorSubcoreMesh(core_axis_name="c", subcore_axis_name="s")


def _segsum_body(h_hbm, src_hbm, dst_hbm, zeros_hbm, out_hbm,
                 src_v, dst_v, rows_v, acc, sem):
  cid = lax.axis_index("c")
  sid = lax.axis_index("s")
  row0 = sid * ROWS_PER_TILE

  pltpu.sync_copy(src_hbm.at[cid, sid], src_v)
  pltpu.sync_copy(dst_hbm.at[cid, sid], dst_v)
  pltpu.sync_copy(zeros_hbm.at[pl.ds(row0, ROWS_PER_TILE)],
                  acc.at[pl.ds(row0, ROWS_PER_TILE)])
  plsc.subcore_barrier()

  def chunk(i, carry):
    pltpu.async_copy(h_hbm.at[src_v.at[i]], rows_v, sem).wait()
    pltpu.sync_copy(rows_v, acc.at[dst_v.at[i]], add=True)
    return carry

  lax.fori_loop(0, CPT, chunk, 0)
  plsc.subcore_barrier()

  pltpu.sync_copy(acc.at[pl.ds(row0, ROWS_PER_TILE)],
                  out_hbm.at[cid, pl.ds(row0, ROWS_PER_TILE)])


_sc_segsum = pl.kernel(
    _segsum_body,
    out_type=jax.ShapeDtypeStruct((NC, N_PAD, D), jnp.float32),
    mesh=_MESH,
    scratch_types=[
        pltpu.VMEM((CPT, CHUNK), jnp.int32),
        pltpu.VMEM((CPT, CHUNK), jnp.int32),
        pltpu.VMEM((CHUNK, D), jnp.float32),
        pltpu.VMEM_SHARED((N_PAD, D), jnp.float32),
        pltpu.SemaphoreType.DMA,
    ],
)


def _cnt_body(dst_hbm, zeros_hbm, ones_hbm, out_hbm, dst_v, ones_v, acc):
  cid = lax.axis_index("c")
  sid = lax.axis_index("s")
  row0 = sid * ROWS_PER_TILE

  pltpu.sync_copy(dst_hbm.at[cid, sid], dst_v)
  pltpu.sync_copy(ones_hbm, ones_v)
  pltpu.sync_copy(zeros_hbm.at[pl.ds(row0, ROWS_PER_TILE)],
                  acc.at[pl.ds(row0, ROWS_PER_TILE)])
  plsc.subcore_barrier()

  def chunk(i, carry):
    pltpu.sync_copy(ones_v, acc.at[dst_v.at[i]], add=True)
    return carry

  lax.fori_loop(0, CPT, chunk, 0)
  plsc.subcore_barrier()
  pltpu.sync_copy(acc.at[pl.ds(row0, ROWS_PER_TILE)],
                  out_hbm.at[cid, pl.ds(row0, ROWS_PER_TILE)])


_sc_cnt = pl.kernel(
    _cnt_body,
    out_type=jax.ShapeDtypeStruct((NC, N_PAD, D), jnp.float32),
    mesh=_MESH,
    scratch_types=[
        pltpu.VMEM((CPT, CHUNK), jnp.int32),
        pltpu.VMEM((CHUNK, D), jnp.float32),
        pltpu.VMEM_SHARED((N_PAD, D), jnp.float32),
    ],
)


def _layer_body(p_ref, c_ref, x_ref, wl_ref, wr_ref, bl_ref, o_ref):
  i = pl.program_id(0)
  msg = p_ref[0] + p_ref[1]
  cnt = c_ref[0][:, 0:1] + c_ref[1][:, 0:1]
  inv = 1.0 / jnp.maximum(cnt, 1.0)
  h = inv * jnp.dot(msg, wl_ref[...], preferred_element_type=jnp.float32)
  h = h + bl_ref[...] + jnp.dot(x_ref[...], wr_ref[...],
                                preferred_element_type=jnp.float32)
  h = jnp.maximum(h, 0.0)
  rowid = i * BLK + lax.broadcasted_iota(jnp.int32, (BLK, 1), 0)
  o_ref[...] = jnp.where(rowid < N_NODES, h, 0.0)


_tc_layer = pl.pallas_call(
    _layer_body,
    grid=(GRID,),
    in_specs=[
        pl.BlockSpec((NC, BLK, D), lambda i: (0, i, 0)),
        pl.BlockSpec((NC, BLK, D), lambda i: (0, i, 0)),
        pl.BlockSpec((BLK, D), lambda i: (i, 0)),
        pl.BlockSpec((D, D), lambda i: (0, 0)),
        pl.BlockSpec((D, D), lambda i: (0, 0)),
        pl.BlockSpec((1, D), lambda i: (0, 0)),
    ],
    out_specs=pl.BlockSpec((BLK, D), lambda i: (i, 0)),
    out_shape=jax.ShapeDtypeStruct((N_PAD, D), jnp.float32),
)


def _final_body(h_ref, b_ref, g_ref, beta_ref, wd_ref, bd_ref, o_ref, pool):
  i = pl.program_id(0)

  @pl.when(i == 0)
  def _():
    pool[...] = jnp.zeros_like(pool)

  b = b_ref[...]
  onehot = (b == lax.broadcasted_iota(jnp.int32, (1, N_GRAPHS), 1)
            ).astype(jnp.float32)
  pool[...] += lax.dot_general(onehot, h_ref[...],
                               (((0,), (0,)), ((), ())),
                               preferred_element_type=jnp.float32)

  @pl.when(i == GRID - 1)
  def _():
    p = pool[...]
    m = jnp.mean(p, axis=1, keepdims=True)
    v = jnp.mean((p - m) * (p - m), axis=1, keepdims=True)
    ln = (p - m) * lax.rsqrt(v + 1e-5) * g_ref[...] + beta_ref[...]
    o_ref[...] = jnp.dot(ln, wd_ref[...],
                         preferred_element_type=jnp.float32) + bd_ref[...]


_tc_final = pl.pallas_call(
    _final_body,
    grid=(GRID,),
    in_specs=[
        pl.BlockSpec((BLK, D), lambda i: (i, 0)),
        pl.BlockSpec((BLK, 1), lambda i: (i, 0)),
        pl.BlockSpec((1, D), lambda i: (0, 0)),
        pl.BlockSpec((1, D), lambda i: (0, 0)),
        pl.BlockSpec((D, D), lambda i: (0, 0)),
        pl.BlockSpec((1, D), lambda i: (0, 0)),
    ],
    out_specs=pl.BlockSpec((N_GRAPHS, D), lambda i: (0, 0)),
    out_shape=jax.ShapeDtypeStruct((N_GRAPHS, D), jnp.float32),
    scratch_shapes=[pltpu.VMEM((N_GRAPHS, D), jnp.float32)],
)


def kernel(x, edge_index, batch, Wl0, Wr0, bl0, Wl1, Wr1, bl1,
           Wl2, Wr2, bl2, ln_g, ln_b, Wd, bd):
  f32 = jnp.float32
  x_pad = jnp.zeros((N_PAD, D), f32).at[:N_NODES].set(x)
  zeros_tab = jnp.zeros((N_PAD, D), f32)
  ones_blk = jnp.ones((CHUNK, D), f32)

  src = jnp.full((E_PAD,), N_NODES, jnp.int32).at[:edge_index.shape[1]].set(
      edge_index[0])
  dst = jnp.full((E_PAD,), N_NODES, jnp.int32).at[:edge_index.shape[1]].set(
      edge_index[1])
  src_r = src.reshape(NC, NS, CPT, CHUNK)
  dst_r = dst.reshape(NC, NS, CPT, CHUNK)

  batch2 = jnp.zeros((N_PAD, 1), jnp.int32).at[:N_NODES, 0].set(batch)

  c0 = _sc_cnt(dst_r, zeros_tab, ones_blk)
  p0 = _sc_segsum(x_pad, src_r, dst_r, zeros_tab)
  h = _tc_layer(p0, c0, x_pad, Wl0.T, Wr0.T, bl0[None])
  p1 = _sc_segsum(h, src_r, dst_r, zeros_tab)
  h = _tc_layer(p1, c0, h, Wl1.T, Wr1.T, bl1[None])
  p2 = _sc_segsum(h, src_r, dst_r, zeros_tab)
  h = _tc_layer(p2, c0, h, Wl2.T, Wr2.T, bl2[None])
  return _tc_final(h, batch2, ln_g[None], ln_b[None], Wd.T, bd[None])


# speedup vs baseline: 3.1813x; 1.2209x over previous
import jax
import jax.numpy as jnp
from jax import lax
from jax.experimental import pallas as pl
from jax.experimental.pallas import tpu as pltpu
from jax.experimental.pallas import tpu_sc as plsc

N_NODES = 10000
N_PAD = 10240
D = 128
N_GRAPHS = 64
NC, NS = 2, 16
NW = NC * NS
CHUNK = 128
CPT = 80
E_PAD = NW * CPT * CHUNK
GRP = 8
CPT0, CPT1 = 128, 32
G0, G1 = CPT0 // GRP, CPT1 // GRP
E0 = NS * CPT0 * CHUNK
ROWS_PER_TILE = N_PAD // NS
BLK = 512
GRID = N_PAD // BLK

_MESH = plsc.VectorSubcoreMesh(core_axis_name="c", subcore_axis_name="s")


def _segsum_body(h_hbm, idx_hbm, zeros_hbm, out_hbm,
                 win_a, win_b, buf_a, buf_b, acc,
                 sem_ia, sem_ib, sem_ga, sem_gb):
  cid = lax.axis_index("c")
  sid = lax.axis_index("s")
  row0 = sid * ROWS_PER_TILE

  pltpu.sync_copy(zeros_hbm.at[pl.ds(row0, ROWS_PER_TILE)],
                  acc.at[pl.ds(row0, ROWS_PER_TILE)])

  ngrp = jnp.where(cid == 0, G0, G1)
  pltpu.async_copy(idx_hbm.at[cid, sid, pl.ds(0, GRP)], win_a, sem_ia)
  pltpu.async_copy(idx_hbm.at[cid, sid, pl.ds(GRP, GRP)], win_b, sem_ib)
  plsc.subcore_barrier()

  bufs = (buf_a, buf_b)
  gsems = (sem_ga, sem_gb)

  def group(g, win, sem_i):
    pltpu.make_async_copy(idx_hbm.at[cid, sid, pl.ds(g * GRP, GRP)],
                          win, sem_i).wait()
    pltpu.async_copy(h_hbm.at[win.at[0, 0]], bufs[0], gsems[0])
    for k in range(GRP):
      if k + 1 < GRP:
        pltpu.async_copy(h_hbm.at[win.at[k + 1, 0]],
                         bufs[(k + 1) % 2], gsems[(k + 1) % 2])
      pltpu.make_async_copy(h_hbm.at[win.at[k, 0]],
                            bufs[k % 2], gsems[k % 2]).wait()
      pltpu.sync_copy(bufs[k % 2], acc.at[win.at[k, 1]], add=True)

    @pl.when(g + 2 < ngrp)
    def _():
      pltpu.async_copy(idx_hbm.at[cid, sid, pl.ds((g + 2) * GRP, GRP)],
                       win, sem_i)

  def pair(j, carry):
    group(2 * j, win_a, sem_ia)
    group(2 * j + 1, win_b, sem_ib)
    return carry

  lax.fori_loop(0, ngrp // 2, pair, 0)
  plsc.subcore_barrier()

  pltpu.sync_copy(acc.at[pl.ds(row0, ROWS_PER_TILE)],
                  out_hbm.at[cid, pl.ds(row0, ROWS_PER_TILE)])


_sc_segsum = pl.kernel(
    _segsum_body,
    out_type=jax.ShapeDtypeStruct((NC, N_PAD, D), jnp.float32),
    mesh=_MESH,
    scratch_types=[
        pltpu.VMEM((GRP, 2, CHUNK), jnp.int32),
        pltpu.VMEM((GRP, 2, CHUNK), jnp.int32),
        pltpu.VMEM((CHUNK, D), jnp.float32),
        pltpu.VMEM((CHUNK, D), jnp.float32),
        pltpu.VMEM_SHARED((N_PAD, D), jnp.float32),
        pltpu.SemaphoreType.DMA,
        pltpu.SemaphoreType.DMA,
        pltpu.SemaphoreType.DMA,
        pltpu.SemaphoreType.DMA,
    ],
)


def _cnt_body(dst_hbm, zeros_hbm, ones_hbm, out_hbm, dst_v, ones_v, acc):
  cid = lax.axis_index("c")
  sid = lax.axis_index("s")
  row0 = sid * ROWS_PER_TILE

  pltpu.sync_copy(dst_hbm.at[cid, sid], dst_v)
  pltpu.sync_copy(ones_hbm, ones_v)
  pltpu.sync_copy(zeros_hbm.at[pl.ds(row0, ROWS_PER_TILE)],
                  acc.at[pl.ds(row0, ROWS_PER_TILE)])
  plsc.subcore_barrier()

  def chunk(i, carry):
    pltpu.sync_copy(ones_v, acc.at[dst_v.at[i]], add=True)
    return carry

  lax.fori_loop(0, CPT, chunk, 0)
  plsc.subcore_barrier()
  pltpu.sync_copy(acc.at[pl.ds(row0, ROWS_PER_TILE)],
                  out_hbm.at[cid, pl.ds(row0, ROWS_PER_TILE)])


_sc_cnt = pl.kernel(
    _cnt_body,
    out_type=jax.ShapeDtypeStruct((NC, N_PAD, D), jnp.float32),
    mesh=_MESH,
    scratch_types=[
        pltpu.VMEM((CPT, CHUNK), jnp.int32),
        pltpu.VMEM((CHUNK, D), jnp.float32),
        pltpu.VMEM_SHARED((N_PAD, D), jnp.float32),
    ],
)


def _layer_body(p_ref, c_ref, x_ref, wl_ref, wr_ref, bl_ref, o_ref):
  i = pl.program_id(0)
  msg = p_ref[0] + p_ref[1]
  cnt = c_ref[0][:, 0:1] + c_ref[1][:, 0:1]
  inv = 1.0 / jnp.maximum(cnt, 1.0)
  h = inv * jnp.dot(msg, wl_ref[...], preferred_element_type=jnp.float32)
  h = h + bl_ref[...] + jnp.dot(x_ref[...], wr_ref[...],
                                preferred_element_type=jnp.float32)
  h = jnp.maximum(h, 0.0)
  rowid = i * BLK + lax.broadcasted_iota(jnp.int32, (BLK, 1), 0)
  o_ref[...] = jnp.where(rowid < N_NODES, h, 0.0)


_tc_layer = pl.pallas_call(
    _layer_body,
    grid=(GRID,),
    in_specs=[
        pl.BlockSpec((NC, BLK, D), lambda i: (0, i, 0)),
        pl.BlockSpec((NC, BLK, D), lambda i: (0, i, 0)),
        pl.BlockSpec((BLK, D), lambda i: (i, 0)),
        pl.BlockSpec((D, D), lambda i: (0, 0)),
        pl.BlockSpec((D, D), lambda i: (0, 0)),
        pl.BlockSpec((1, D), lambda i: (0, 0)),
    ],
    out_specs=pl.BlockSpec((BLK, D), lambda i: (i, 0)),
    out_shape=jax.ShapeDtypeStruct((N_PAD, D), jnp.float32),
)


def _final_body(h_ref, b_ref, g_ref, beta_ref, wd_ref, bd_ref, o_ref, pool):
  i = pl.program_id(0)

  @pl.when(i == 0)
  def _():
    pool[...] = jnp.zeros_like(pool)

  b = b_ref[...]
  onehot = (b == lax.broadcasted_iota(jnp.int32, (1, N_GRAPHS), 1)
            ).astype(jnp.float32)
  pool[...] += lax.dot_general(onehot, h_ref[...],
                               (((0,), (0,)), ((), ())),
                               preferred_element_type=jnp.float32)

  @pl.when(i == GRID - 1)
  def _():
    p = pool[...]
    m = jnp.mean(p, axis=1, keepdims=True)
    v = jnp.mean((p - m) * (p - m), axis=1, keepdims=True)
    ln = (p - m) * lax.rsqrt(v + 1e-5) * g_ref[...] + beta_ref[...]
    o_ref[...] = jnp.dot(ln, wd_ref[...],
                         preferred_element_type=jnp.float32) + bd_ref[...]


_tc_final = pl.pallas_call(
    _final_body,
    grid=(GRID,),
    in_specs=[
        pl.BlockSpec((BLK, D), lambda i: (i, 0)),
        pl.BlockSpec((BLK, 1), lambda i: (i, 0)),
        pl.BlockSpec((1, D), lambda i: (0, 0)),
        pl.BlockSpec((1, D), lambda i: (0, 0)),
        pl.BlockSpec((D, D), lambda i: (0, 0)),
        pl.BlockSpec((1, D), lambda i: (0, 0)),
    ],
    out_specs=pl.BlockSpec((N_GRAPHS, D), lambda i: (0, 0)),
    out_shape=jax.ShapeDtypeStruct((N_GRAPHS, D), jnp.float32),
    scratch_shapes=[pltpu.VMEM((N_GRAPHS, D), jnp.float32)],
)


def kernel(x, edge_index, batch, Wl0, Wr0, bl0, Wl1, Wr1, bl1,
           Wl2, Wr2, bl2, ln_g, ln_b, Wd, bd):
  f32 = jnp.float32
  x_pad = jnp.zeros((N_PAD, D), f32).at[:N_NODES].set(x)
  zeros_tab = jnp.zeros((N_PAD, D), f32)
  ones_blk = jnp.ones((CHUNK, D), f32)

  src = jnp.full((E_PAD,), N_NODES, jnp.int32).at[:edge_index.shape[1]].set(
      edge_index[0])
  dst = jnp.full((E_PAD,), N_NODES, jnp.int32).at[:edge_index.shape[1]].set(
      edge_index[1])
  def core_idx(s, d):
    return jnp.stack([s, d], axis=2)

  pad_tail = jnp.full((NS, CPT0 - CPT1, 2, CHUNK), N_NODES, jnp.int32)
  idx0 = core_idx(src[:E0].reshape(NS, CPT0, CHUNK),
                  dst[:E0].reshape(NS, CPT0, CHUNK))
  idx1 = jnp.concatenate([
      core_idx(src[E0:].reshape(NS, CPT1, CHUNK),
               dst[E0:].reshape(NS, CPT1, CHUNK)), pad_tail], axis=1)
  idx = jnp.stack([idx0, idx1])
  dst_r = dst.reshape(NC, NS, CPT, CHUNK)

  batch2 = jnp.zeros((N_PAD, 1), jnp.int32).at[:N_NODES, 0].set(batch)

  c0 = _sc_cnt(dst_r, zeros_tab, ones_blk)
  x_dep, _ = lax.optimization_barrier((x_pad, c0))
  p0 = _sc_segsum(x_dep, idx, zeros_tab)
  h = _tc_layer(p0, c0, x_pad, Wl0.T, Wr0.T, bl0[None])
  p1 = _sc_segsum(h, idx, zeros_tab)
  h = _tc_layer(p1, c0, h, Wl1.T, Wr1.T, bl1[None])
  p2 = _sc_segsum(h, idx, zeros_tab)
  h = _tc_layer(p2, c0, h, Wl2.T, Wr2.T, bl2[None])
  return _tc_final(h, batch2, ln_g[None], ln_b[None], Wd.T, bd[None])


# speedup vs baseline: 3.7042x; 1.1643x over previous
import jax
import jax.numpy as jnp
from jax import lax
from jax.experimental import pallas as pl
from jax.experimental.pallas import tpu as pltpu
from jax.experimental.pallas import tpu_sc as plsc

N_NODES = 10000
N_PAD = 10240
D = 128
N_GRAPHS = 64
NC, NS = 2, 16
NW = NC * NS
CHUNK = 128
CPT = 80
E_PAD = NW * CPT * CHUNK
GRP = 8
CPT0, CPT1 = 144, 16
G0, G1 = CPT0 // GRP, CPT1 // GRP
E0 = NS * CPT0 * CHUNK
ROWS_PER_TILE = N_PAD // NS
BLK = 512
GRID = N_PAD // BLK

_MESH = plsc.VectorSubcoreMesh(core_axis_name="c", subcore_axis_name="s")


def _segsum_body(h_hbm, idx_hbm, zeros_hbm, out_hbm,
                 win_a, win_b, buf_a, buf_b, acc,
                 sem_ia, sem_ib, sem_ga, sem_gb):
  cid = lax.axis_index("c")
  sid = lax.axis_index("s")
  row0 = sid * ROWS_PER_TILE

  pltpu.sync_copy(zeros_hbm.at[pl.ds(row0, ROWS_PER_TILE)],
                  acc.at[pl.ds(row0, ROWS_PER_TILE)])

  ngrp = jnp.where(cid == 0, G0, G1)
  pltpu.async_copy(idx_hbm.at[cid, sid, pl.ds(0, GRP)], win_a, sem_ia)
  pltpu.async_copy(idx_hbm.at[cid, sid, pl.ds(GRP, GRP)], win_b, sem_ib)
  plsc.subcore_barrier()

  bufs = (buf_a, buf_b)
  gsems = (sem_ga, sem_gb)

  def group(g, win, sem_i):
    pltpu.make_async_copy(idx_hbm.at[cid, sid, pl.ds(g * GRP, GRP)],
                          win, sem_i).wait()
    pltpu.async_copy(h_hbm.at[win.at[0, 0]], bufs[0], gsems[0])
    for k in range(GRP):
      if k + 1 < GRP:
        pltpu.async_copy(h_hbm.at[win.at[k + 1, 0]],
                         bufs[(k + 1) % 2], gsems[(k + 1) % 2])
      pltpu.make_async_copy(h_hbm.at[win.at[k, 0]],
                            bufs[k % 2], gsems[k % 2]).wait()
      pltpu.sync_copy(bufs[k % 2], acc.at[win.at[k, 1]], add=True)

    @pl.when(g + 2 < ngrp)
    def _():
      pltpu.async_copy(idx_hbm.at[cid, sid, pl.ds((g + 2) * GRP, GRP)],
                       win, sem_i)

  def pair(j, carry):
    group(2 * j, win_a, sem_ia)
    group(2 * j + 1, win_b, sem_ib)
    return carry

  lax.fori_loop(0, ngrp // 2, pair, 0)
  plsc.subcore_barrier()

  pltpu.sync_copy(acc.at[pl.ds(row0, ROWS_PER_TILE)],
                  out_hbm.at[cid, pl.ds(row0, ROWS_PER_TILE)])


_sc_segsum = pl.kernel(
    _segsum_body,
    out_type=jax.ShapeDtypeStruct((NC, N_PAD, D), jnp.float32),
    mesh=_MESH,
    scratch_types=[
        pltpu.VMEM((GRP, 2, CHUNK), jnp.int32),
        pltpu.VMEM((GRP, 2, CHUNK), jnp.int32),
        pltpu.VMEM((CHUNK, D), jnp.float32),
        pltpu.VMEM((CHUNK, D), jnp.float32),
        pltpu.VMEM_SHARED((N_PAD, D), jnp.float32),
        pltpu.SemaphoreType.DMA,
        pltpu.SemaphoreType.DMA,
        pltpu.SemaphoreType.DMA,
        pltpu.SemaphoreType.DMA,
    ],
)


def _cnt_body(dst_hbm, zeros_hbm, ones_hbm, out_hbm, dst_v, ones_v, acc):
  cid = lax.axis_index("c")
  sid = lax.axis_index("s")
  row0 = sid * ROWS_PER_TILE

  pltpu.sync_copy(dst_hbm.at[cid, sid], dst_v)
  pltpu.sync_copy(ones_hbm, ones_v)
  pltpu.sync_copy(zeros_hbm.at[pl.ds(row0, ROWS_PER_TILE)],
                  acc.at[pl.ds(row0, ROWS_PER_TILE)])
  plsc.subcore_barrier()

  def chunk(i, carry):
    pltpu.sync_copy(ones_v, acc.at[dst_v.at[i]], add=True)
    return carry

  lax.fori_loop(0, CPT, chunk, 0)
  plsc.subcore_barrier()
  pltpu.sync_copy(acc.at[pl.ds(row0, ROWS_PER_TILE)],
                  out_hbm.at[cid, pl.ds(row0, ROWS_PER_TILE)])


_sc_cnt = pl.kernel(
    _cnt_body,
    out_type=jax.ShapeDtypeStruct((NC, N_PAD, D), jnp.float32),
    mesh=_MESH,
    scratch_types=[
        pltpu.VMEM((CPT, CHUNK), jnp.int32),
        pltpu.VMEM((CHUNK, D), jnp.float32),
        pltpu.VMEM_SHARED((N_PAD, D), jnp.float32),
    ],
)


def _layer_body(p_ref, c_ref, x_ref, wl_ref, wr_ref, bl_ref, o_ref):
  i = pl.program_id(0)
  msg = p_ref[0] + p_ref[1]
  cnt = c_ref[0][:, 0:1] + c_ref[1][:, 0:1]
  inv = 1.0 / jnp.maximum(cnt, 1.0)
  h = inv * jnp.dot(msg, wl_ref[...], preferred_element_type=jnp.float32)
  h = h + bl_ref[...] + jnp.dot(x_ref[...], wr_ref[...],
                                preferred_element_type=jnp.float32)
  h = jnp.maximum(h, 0.0)
  rowid = i * BLK + lax.broadcasted_iota(jnp.int32, (BLK, 1), 0)
  o_ref[...] = jnp.where(rowid < N_NODES, h, 0.0)


_tc_layer = pl.pallas_call(
    _layer_body,
    grid=(GRID,),
    in_specs=[
        pl.BlockSpec((NC, BLK, D), lambda i: (0, i, 0)),
        pl.BlockSpec((NC, BLK, D), lambda i: (0, i, 0)),
        pl.BlockSpec((BLK, D), lambda i: (i, 0)),
        pl.BlockSpec((D, D), lambda i: (0, 0)),
        pl.BlockSpec((D, D), lambda i: (0, 0)),
        pl.BlockSpec((1, D), lambda i: (0, 0)),
    ],
    out_specs=pl.BlockSpec((BLK, D), lambda i: (i, 0)),
    out_shape=jax.ShapeDtypeStruct((N_PAD, D), jnp.float32),
)


def _final_body(h_ref, b_ref, g_ref, beta_ref, wd_ref, bd_ref, o_ref, pool):
  i = pl.program_id(0)

  @pl.when(i == 0)
  def _():
    pool[...] = jnp.zeros_like(pool)

  b = b_ref[...]
  onehot = (b == lax.broadcasted_iota(jnp.int32, (1, N_GRAPHS), 1)
            ).astype(jnp.float32)
  pool[...] += lax.dot_general(onehot, h_ref[...],
                               (((0,), (0,)), ((), ())),
                               preferred_element_type=jnp.float32)

  @pl.when(i == GRID - 1)
  def _():
    p = pool[...]
    m = jnp.mean(p, axis=1, keepdims=True)
    v = jnp.mean((p - m) * (p - m), axis=1, keepdims=True)
    ln = (p - m) * lax.rsqrt(v + 1e-5) * g_ref[...] + beta_ref[...]
    o_ref[...] = jnp.dot(ln, wd_ref[...],
                         preferred_element_type=jnp.float32) + bd_ref[...]


_tc_final = pl.pallas_call(
    _final_body,
    grid=(GRID,),
    in_specs=[
        pl.BlockSpec((BLK, D), lambda i: (i, 0)),
        pl.BlockSpec((BLK, 1), lambda i: (i, 0)),
        pl.BlockSpec((1, D), lambda i: (0, 0)),
        pl.BlockSpec((1, D), lambda i: (0, 0)),
        pl.BlockSpec((D, D), lambda i: (0, 0)),
        pl.BlockSpec((1, D), lambda i: (0, 0)),
    ],
    out_specs=pl.BlockSpec((N_GRAPHS, D), lambda i: (0, 0)),
    out_shape=jax.ShapeDtypeStruct((N_GRAPHS, D), jnp.float32),
    scratch_shapes=[pltpu.VMEM((N_GRAPHS, D), jnp.float32)],
)


def kernel(x, edge_index, batch, Wl0, Wr0, bl0, Wl1, Wr1, bl1,
           Wl2, Wr2, bl2, ln_g, ln_b, Wd, bd):
  f32 = jnp.float32
  x_pad = jnp.zeros((N_PAD, D), f32).at[:N_NODES].set(x)
  zeros_tab = jnp.zeros((N_PAD, D), f32)
  ones_blk = jnp.ones((CHUNK, D), f32)

  src = jnp.full((E_PAD,), N_NODES, jnp.int32).at[:edge_index.shape[1]].set(
      edge_index[0])
  dst = jnp.full((E_PAD,), N_NODES, jnp.int32).at[:edge_index.shape[1]].set(
      edge_index[1])
  def core_idx(s, d):
    return jnp.stack([s, d], axis=2)

  pad_tail = jnp.full((NS, CPT0 - CPT1, 2, CHUNK), N_NODES, jnp.int32)
  idx0 = core_idx(src[:E0].reshape(NS, CPT0, CHUNK),
                  dst[:E0].reshape(NS, CPT0, CHUNK))
  idx1 = jnp.concatenate([
      core_idx(src[E0:].reshape(NS, CPT1, CHUNK),
               dst[E0:].reshape(NS, CPT1, CHUNK)), pad_tail], axis=1)
  idx = jnp.stack([idx0, idx1])
  dst_r = dst.reshape(NC, NS, CPT, CHUNK)

  batch2 = jnp.zeros((N_PAD, 1), jnp.int32).at[:N_NODES, 0].set(batch)

  c0 = _sc_cnt(dst_r, zeros_tab, ones_blk)
  x_dep, _ = lax.optimization_barrier((x_pad, c0))
  p0 = _sc_segsum(x_dep, idx, zeros_tab)
  h = _tc_layer(p0, c0, x_pad, Wl0.T, Wr0.T, bl0[None])
  p1 = _sc_segsum(h, idx, zeros_tab)
  h = _tc_layer(p1, c0, h, Wl1.T, Wr1.T, bl1[None])
  p2 = _sc_segsum(h, idx, zeros_tab)
  h = _tc_layer(p2, c0, h, Wl2.T, Wr2.T, bl2[None])
  return _tc_final(h, batch2, ln_g[None], ln_b[None], Wd.T, bd[None])
